# baseline (speedup 1.0000x reference)
import functools

import jax
import jax.numpy as jnp
from jax import lax
from jax.experimental import pallas as pl
from jax.experimental.pallas import tpu as pltpu
from jax.experimental.pallas import tpu_sc as plsc

N_NODES = 10000
N_EDGES = 320000
D = 128
DH = 64

NC = 2
NS = 16
EDGES_PER_TILE = N_EDGES // NS
CHUNK = 125
NCHUNK = EDGES_PER_TILE // CHUNK
TAIL0 = CHUNK - 16
NPAD = 10240
RPT = NPAD // NS
HR = NPAD // 16
XV_LEN = 2 * N_NODES - 1

_sc_mesh = plsc.VectorSubcoreMesh(
    core_axis_name="c", subcore_axis_name="s", num_cores=NC, num_subcores=NS
)


@functools.partial(
    pl.kernel,
    out_type=(
        jax.ShapeDtypeStruct((NC, NPAD, DH), jnp.float32),
        jax.ShapeDtypeStruct((HR, 16), jnp.float32),
    ),
    mesh=_sc_mesh,
    scratch_types=[
        [pltpu.VMEM((16, CHUNK), jnp.int32)] * 2,
        pltpu.VMEM((8, CHUNK, DH), jnp.float32),
        pltpu.VMEM((HR, 16), jnp.float32),
        pltpu.VMEM((HR // 128, 128), jnp.int32),
        pltpu.VMEM_SHARED((NPAD, DH), jnp.float32),
        pltpu.VMEM_SHARED((HR, 16), jnp.float32),
        pltpu.SemaphoreType.DMA,
        pltpu.SemaphoreType.DMA,
        pltpu.SemaphoreType.DMA,
    ],
    compiler_params=pltpu.CompilerParams(
        use_tc_tiling_on_sc=False, needs_layout_passes=False
    ),
)
def _aggregate(x_hbm, edges_hbm, zrows_hbm, zhist_hbm, iota_hbm,
               outf_hbm, outd_hbm,
               idx_ring, rows_v, hist_v, iota_v,
               acc_sh, deg_sh, isem, gsem, ssem):
    cid = lax.axis_index("c")
    sid = lax.axis_index("s")
    xp = x_hbm.at[pl.ds(cid, XV_LEN)]

    pltpu.sync_copy(iota_hbm, iota_v)

    pltpu.sync_copy(zrows_hbm, acc_sh.at[pl.ds(sid * RPT, RPT)])
    pltpu.sync_copy(zhist_hbm, hist_v)
    pltpu.sync_copy(
        zhist_hbm.at[pl.ds(0, RPT // 16)],
        deg_sh.at[pl.ds(sid * (RPT // 16), RPT // 16)],
    )
    plsc.subcore_barrier()

    ones16 = jnp.full((16,), 1.0, jnp.float32)
    tailmask = jnp.arange(16, dtype=jnp.int32) >= (112 - TAIL0)

    NBLK = NCHUNK // 8
    gbase = sid * NBLK

    def stage(g, slot):
        pltpu.async_copy(edges_hbm.at[gbase + g], idx_ring[slot], isem)

    def stage_wait(slot):
        pltpu.make_async_copy(edges_hbm.at[gbase], idx_ring[slot], isem).wait()

    stage(0, 0)
    stage(1, 1)
    stage_wait(0)
    stage_wait(1)
    for k in range(8):
        pltpu.async_copy(xp.at[idx_ring[0].at[2 * k]], rows_v.at[k], gsem)

    def hist_row(slot, k):
        for q in range(7):
            dd = idx_ring[slot][2 * k + 1, pl.ds(q * 16, 16)]
            plsc.addupdate_scatter(hist_v, [dd >> 4, dd & 15], ones16)
        dd = idx_ring[slot][2 * k + 1, pl.ds(TAIL0, 16)]
        plsc.addupdate_scatter(hist_v, [dd >> 4, dd & 15], ones16, mask=tailmask)

    def body(jj, _):
        for half in (0, 1):
            g = 2 * jj + half
            cur = half
            nxt = 1 - half

            @pl.when((g >= 1) & (g + 1 < NBLK))
            def _():
                stage_wait(nxt)

            for k in range(8):
                t = 8 * g + k
                pltpu.make_async_copy(
                    xp.at[idx_ring[cur].at[2 * k]], rows_v.at[k], gsem
                ).wait()
                d = pltpu.async_copy(
                    rows_v.at[k],
                    acc_sh.at[idx_ring[cur].at[2 * k + 1]],
                    ssem,
                    add=True,
                )
                hist_row(cur, k)
                d.wait()

                @pl.when(t + 8 < NCHUNK)
                def _():
                    pltpu.async_copy(
                        xp.at[idx_ring[nxt].at[2 * k]], rows_v.at[k], gsem
                    )

            @pl.when(g + 2 < NBLK)
            def _():
                stage(g + 2, cur)

        return 0

    lax.fori_loop(0, NBLK // 2, body, 0)


    for c5 in range(HR // 128):
        pltpu.sync_copy(
            hist_v.at[pl.ds(c5 * 128, 128)],
            deg_sh.at[iota_v.at[c5]],
            add=True,
        )

    plsc.subcore_barrier()

    row0 = sid * RPT
    pltpu.sync_copy(acc_sh.at[pl.ds(row0, RPT)], outf_hbm.at[cid, pl.ds(row0, RPT)])

    @pl.when((cid == 0) & (sid == 0))
    def _():
        pltpu.sync_copy(deg_sh, outd_hbm)


NPAIR = N_NODES // 2
PBLK = 1000


def _finish_body(pv_ref, dv_ref, w_ref, b_ref, out_ref):
    a = pv_ref[0]
    bb = pv_ref[1]
    h_e = jnp.concatenate([a[:, :DH], bb[:, :DH]], axis=1)
    h_o = jnp.concatenate([a[:, DH:], bb[:, DH:]], axis=1)
    inv = 1.0 / jnp.maximum(dv_ref[...], 1.0)
    w = w_ref[...]
    dims = (((1,), (1,)), ((), ()))
    o_e = (
        lax.dot_general(h_e, w, dims, preferred_element_type=jnp.float32)
        * inv[:, 0:1] + b_ref[...] + 0.01
    )
    o_o = (
        lax.dot_general(h_o, w, dims, preferred_element_type=jnp.float32)
        * inv[:, 1:2] + b_ref[...] + 0.01
    )
    pair = jnp.concatenate([o_e[:, None, :], o_o[:, None, :]], axis=1)
    out_ref[...] = pair.reshape(2 * PBLK, D)


_finish = pl.pallas_call(
    _finish_body,
    grid=(NPAIR // PBLK,),
    in_specs=[
        pl.BlockSpec((NC, PBLK, D), lambda i: (0, i, 0)),
        pl.BlockSpec((PBLK, 2), lambda i: (i, 0)),
        pl.BlockSpec((D, D), lambda i: (0, 0)),
        pl.BlockSpec((1, D), lambda i: (0, 0)),
    ],
    out_specs=pl.BlockSpec((2 * PBLK, D), lambda i: (i, 0)),
    out_shape=jax.ShapeDtypeStruct((N_NODES, D), jnp.float32),
)


@jax.jit
def kernel(x, edge_index, W_neigh, b_neigh):
    xv = x.reshape(2 * N_NODES, DH)
    mul = jnp.array([2, 1], jnp.int32).reshape(2, 1, 1, 1)
    edges = (
        (edge_index.reshape(2, NS, NCHUNK, CHUNK) * mul)
        .transpose(1, 2, 0, 3)
        .reshape(NS * (NCHUNK // 8), 16, CHUNK)
    )
    zrows = jnp.zeros((RPT, DH), jnp.float32)
    zhist = jnp.zeros((HR, 16), jnp.float32)
    iota = jnp.arange(HR, dtype=jnp.int32).reshape(HR // 128, 128)
    parts, deg = _aggregate(xv, edges, zrows, zhist, iota)
    pv = parts.reshape(NC, NPAD // 2, D)
    dv = deg.reshape(NPAD // 2, 2)[:NPAIR]
    return _finish(pv, dv, W_neigh, b_neigh.reshape(1, D))


# speedup vs baseline: 1.0770x; 1.0770x over previous
import functools

import jax
import jax.numpy as jnp
from jax import lax
from jax.experimental import pallas as pl
from jax.experimental.pallas import tpu as pltpu
from jax.experimental.pallas import tpu_sc as plsc

N_NODES = 10000
N_EDGES = 320000
D = 128
DH = 64

NC = 2
NS = 16
EDGES_PER_TILE = N_EDGES // NS
CHUNK = 125
NCHUNK = EDGES_PER_TILE // CHUNK
TAIL0 = CHUNK - 16
NPAD = 10240
RPT = NPAD // NS
HR = NPAD // 16
XV_LEN = 2 * N_NODES - 1

_sc_mesh = plsc.VectorSubcoreMesh(
    core_axis_name="c", subcore_axis_name="s", num_cores=NC, num_subcores=NS
)


@functools.partial(
    pl.kernel,
    out_type=(
        jax.ShapeDtypeStruct((NC, NPAD, DH), jnp.float32),
        jax.ShapeDtypeStruct((HR, 16), jnp.float32),
    ),
    mesh=_sc_mesh,
    scratch_types=[
        pltpu.VMEM((NCHUNK, CHUNK), jnp.int32),
        pltpu.VMEM((NCHUNK, CHUNK), jnp.int32),
        pltpu.VMEM((4, CHUNK, DH), jnp.float32),
        pltpu.VMEM((HR, 16), jnp.float32),
        pltpu.VMEM((HR // 128, 128), jnp.int32),
        pltpu.VMEM_SHARED((NPAD, DH), jnp.float32),
        pltpu.VMEM_SHARED((HR, 16), jnp.float32),
        pltpu.SemaphoreType.DMA,
        pltpu.SemaphoreType.DMA,
        pltpu.SemaphoreType.DMA,
        pltpu.SemaphoreType.DMA,
        pltpu.SemaphoreType.DMA,
    ],
    compiler_params=pltpu.CompilerParams(
        use_tc_tiling_on_sc=False, needs_layout_passes=False
    ),
)
def _aggregate(x_hbm, edges_hbm, zrows_hbm, zhist_hbm, iota_hbm,
               outf_hbm, outd_hbm,
               src_v, dst_v, rows_v, hist_v, iota_v,
               acc_sh, deg_sh, gsem0, gsem1, gsem2, gsem3, ssem):
    cid = lax.axis_index("c")
    sid = lax.axis_index("s")
    xp = x_hbm.at[pl.ds(cid, XV_LEN)]

    pltpu.sync_copy(edges_hbm.at[0, sid], src_v)
    pltpu.sync_copy(edges_hbm.at[1, sid], dst_v)
    pltpu.sync_copy(iota_hbm, iota_v)

    pltpu.sync_copy(zrows_hbm, acc_sh.at[pl.ds(sid * RPT, RPT)])
    pltpu.sync_copy(zhist_hbm, hist_v)
    pltpu.sync_copy(
        zhist_hbm.at[pl.ds(0, RPT // 16)],
        deg_sh.at[pl.ds(sid * (RPT // 16), RPT // 16)],
    )
    plsc.subcore_barrier()

    gsems = (gsem0, gsem1, gsem2, gsem3)
    ones16 = jnp.full((16,), 1.0, jnp.float32)
    tailmask = jnp.arange(16, dtype=jnp.int32) >= (112 - TAIL0)

    for b in range(3):
        pltpu.async_copy(xp.at[src_v.at[b]], rows_v.at[b], gsems[b % 4])

    def hist_row(t):
        for k in range(7):
            dd = dst_v[t, pl.ds(k * 16, 16)]
            plsc.addupdate_scatter(hist_v, [dd >> 4, dd & 15], ones16)
        dd = dst_v[t, pl.ds(TAIL0, 16)]
        plsc.addupdate_scatter(hist_v, [dd >> 4, dd & 15], ones16, mask=tailmask)

    def scatter_wait(t):
        pltpu.make_async_copy(rows_v.at[0], acc_sh.at[dst_v.at[t]], ssem).wait()

    def body(jj, _):
        for b in range(4):
            t = 4 * jj + b
            pltpu.make_async_copy(
                xp.at[src_v.at[t]], rows_v.at[b], gsems[b]
            ).wait()
            pltpu.async_copy(rows_v.at[b], acc_sh.at[dst_v.at[t]], ssem, add=True)
            hist_row(t)

            @pl.when(t >= 1)
            def _():
                scatter_wait(t - 1)

            @pl.when(t + 3 < NCHUNK)
            def _():
                pltpu.async_copy(
                    xp.at[src_v.at[t + 3]], rows_v.at[(b + 3) % 4],
                    gsems[(b + 3) % 4],
                )

        return 0

    lax.fori_loop(0, NCHUNK // 4, body, 0)
    scatter_wait(NCHUNK - 1)

    for c5 in range(HR // 128):
        pltpu.sync_copy(
            hist_v.at[pl.ds(c5 * 128, 128)],
            deg_sh.at[iota_v.at[c5]],
            add=True,
        )

    plsc.subcore_barrier()

    row0 = sid * RPT
    pltpu.sync_copy(acc_sh.at[pl.ds(row0, RPT)], outf_hbm.at[cid, pl.ds(row0, RPT)])

    @pl.when((cid == 0) & (sid == 0))
    def _():
        pltpu.sync_copy(deg_sh, outd_hbm)


NPAIR = N_NODES // 2
PBLK = 1000


def _finish_body(pv_ref, dv_ref, w_ref, b_ref, out_ref):
    a = pv_ref[0]
    bb = pv_ref[1]
    h_e = jnp.concatenate([a[:, :DH], bb[:, :DH]], axis=1)
    h_o = jnp.concatenate([a[:, DH:], bb[:, DH:]], axis=1)
    inv = 1.0 / jnp.maximum(dv_ref[...], 1.0)
    w = w_ref[...]
    dims = (((1,), (1,)), ((), ()))
    o_e = (
        lax.dot_general(h_e, w, dims, preferred_element_type=jnp.float32)
        * inv[:, 0:1] + b_ref[...] + 0.01
    )
    o_o = (
        lax.dot_general(h_o, w, dims, preferred_element_type=jnp.float32)
        * inv[:, 1:2] + b_ref[...] + 0.01
    )
    pair = jnp.concatenate([o_e[:, None, :], o_o[:, None, :]], axis=1)
    out_ref[...] = pair.reshape(2 * PBLK, D)


_finish = pl.pallas_call(
    _finish_body,
    grid=(NPAIR // PBLK,),
    in_specs=[
        pl.BlockSpec((NC, PBLK, D), lambda i: (0, i, 0)),
        pl.BlockSpec((PBLK, 2), lambda i: (i, 0)),
        pl.BlockSpec((D, D), lambda i: (0, 0)),
        pl.BlockSpec((1, D), lambda i: (0, 0)),
    ],
    out_specs=pl.BlockSpec((2 * PBLK, D), lambda i: (i, 0)),
    out_shape=jax.ShapeDtypeStruct((N_NODES, D), jnp.float32),
)


@jax.jit
def kernel(x, edge_index, W_neigh, b_neigh):
    xv = x.reshape(2 * N_NODES, DH)
    em = edge_index * jnp.array([[2], [1]], jnp.int32)
    edges = em.reshape(2, NS, NCHUNK, CHUNK)
    zrows = jnp.zeros((RPT, DH), jnp.float32)
    zhist = jnp.zeros((HR, 16), jnp.float32)
    iota = jnp.arange(HR, dtype=jnp.int32).reshape(HR // 128, 128)
    parts, deg = _aggregate(xv, edges, zrows, zhist, iota)
    pv = parts.reshape(NC, NPAD // 2, D)
    dv = deg.reshape(NPAD // 2, 2)[:NPAIR]
    return _finish(pv, dv, W_neigh, b_neigh.reshape(1, D))
